# initial kernel scaffold (unmeasured)
import jax
import jax.numpy as jnp
from jax import lax
from jax.experimental import pallas as pl
from jax.experimental.pallas import tpu as pltpu


def kernel(
    x,
):
    def body(*refs):
        pass

    out_shape = jax.ShapeDtypeStruct(..., jnp.float32)
    return pl.pallas_call(body, out_shape=out_shape)(...)


# baseline (device time: 903632 ns/iter reference)
import jax
import jax.numpy as jnp
from jax import lax
from jax.experimental import pallas as pl
from jax.experimental.pallas import tpu as pltpu

M = 32768
N = 1024
CONV_CH = 2048


def kernel(x):
    assert x.shape == (M, N), x.shape

    def body(x_hbm, out_hbm, f32_vmem, bf16_vmem, conv_sem, send_sem, recv_sem):
        my_x = lax.axis_index("x")
        my_y = lax.axis_index("y")
        row0 = my_x * M

        for i in range(M // CONV_CH):
            cp_in = pltpu.make_async_copy(
                x_hbm.at[pl.ds(i * CONV_CH, CONV_CH)], f32_vmem, conv_sem
            )
            cp_in.start()
            cp_in.wait()
            bf16_vmem[...] = f32_vmem[...].astype(jnp.bfloat16)
            cp_out = pltpu.make_async_copy(
                bf16_vmem,
                out_hbm.at[pl.ds(row0 + i * CONV_CH, CONV_CH)],
                conv_sem,
            )
            cp_out.start()
            cp_out.wait()

        rdma = pltpu.make_async_remote_copy(
            src_ref=out_hbm.at[pl.ds(row0, M)],
            dst_ref=out_hbm.at[pl.ds(row0, M)],
            send_sem=send_sem,
            recv_sem=recv_sem,
            device_id=(1 - my_x, my_y),
            device_id_type=pl.DeviceIdType.MESH,
        )
        rdma.start()
        rdma.wait()

    return pl.pallas_call(
        body,
        out_shape=jax.ShapeDtypeStruct((2 * M, N), jnp.bfloat16),
        in_specs=[pl.BlockSpec(memory_space=pl.ANY)],
        out_specs=pl.BlockSpec(memory_space=pl.ANY),
        scratch_shapes=[
            pltpu.VMEM((CONV_CH, N), jnp.float32),
            pltpu.VMEM((CONV_CH, N), jnp.bfloat16),
            pltpu.SemaphoreType.DMA,
            pltpu.SemaphoreType.DMA,
            pltpu.SemaphoreType.DMA,
        ],
    )(x)


# device time: 490988 ns/iter; 1.8404x vs baseline; 1.8404x over previous
import jax
import jax.numpy as jnp
from jax import lax
from jax.experimental import pallas as pl
from jax.experimental.pallas import tpu as pltpu

M = 32768
N = 1024
H = M // 2
NC = 16
CH = H // NC


def kernel(x):
    assert x.shape == (M, N), x.shape

    def body(x_hbm, out_hbm, f32_v, bf16_v, cin_sems, cout_sems,
             ssx, rsx, ssy, rsy):
        my_x = lax.axis_index("x")
        my_y = lax.axis_index("y")
        row0 = my_x * M
        peer_row0 = (1 - my_x) * M
        send_half = my_y * H
        other_half = (1 - my_y) * H
        x_nbr = (1 - my_x, my_y)
        y_nbr = (my_x, 1 - my_y)

        pending_in = {}

        def conv_start(local_r, slot):
            cp = pltpu.make_async_copy(
                x_hbm.at[pl.ds(local_r, CH)], f32_v.at[slot], cin_sems.at[slot]
            )
            cp.start()
            pending_in[slot] = cp

        def conv_finish_store(local_r, slot):
            pending_in.pop(slot).wait()
            bf16_v[slot] = f32_v[slot].astype(jnp.bfloat16)
            cp = pltpu.make_async_copy(
                bf16_v.at[slot],
                out_hbm.at[pl.ds(row0 + local_r, CH)],
                cout_sems.at[slot],
            )
            cp.start()
            cp.wait()

        x_rdma = []
        conv_start(send_half, 0)
        for c in range(NC):
            if c + 1 < NC:
                conv_start(send_half + (c + 1) * CH, (c + 1) % 2)
            conv_finish_store(send_half + c * CH, c % 2)
            r = out_hbm.at[pl.ds(row0 + send_half + c * CH, CH)]
            rdma = pltpu.make_async_remote_copy(
                src_ref=r, dst_ref=r,
                send_sem=ssx.at[c], recv_sem=rsx.at[c],
                device_id=x_nbr, device_id_type=pl.DeviceIdType.MESH,
            )
            rdma.start()
            x_rdma.append(rdma)

        fwd = []
        conv_start(other_half, 0)
        if NC > 1:
            conv_start(other_half + CH, 1)
        for c in range(NC):
            x_rdma[c].wait_recv()
            r = out_hbm.at[pl.ds(peer_row0 + send_half + c * CH, CH)]
            f = pltpu.make_async_remote_copy(
                src_ref=r, dst_ref=r,
                send_sem=ssy.at[c], recv_sem=rsy.at[c],
                device_id=y_nbr, device_id_type=pl.DeviceIdType.MESH,
            )
            f.start()
            fwd.append(f)
            conv_finish_store(other_half + c * CH, c % 2)
            if c + 2 < NC:
                conv_start(other_half + (c + 2) * CH, c % 2)

        for c in range(NC):
            fwd[c].wait_recv()
        for c in range(NC):
            x_rdma[c].wait_send()
            fwd[c].wait_send()

    return pl.pallas_call(
        body,
        out_shape=jax.ShapeDtypeStruct((2 * M, N), jnp.bfloat16),
        in_specs=[pl.BlockSpec(memory_space=pl.ANY)],
        out_specs=pl.BlockSpec(memory_space=pl.ANY),
        scratch_shapes=[
            pltpu.VMEM((2, CH, N), jnp.float32),
            pltpu.VMEM((2, CH, N), jnp.bfloat16),
            pltpu.SemaphoreType.DMA((2,)),
            pltpu.SemaphoreType.DMA((2,)),
            pltpu.SemaphoreType.DMA((NC,)),
            pltpu.SemaphoreType.DMA((NC,)),
            pltpu.SemaphoreType.DMA((NC,)),
            pltpu.SemaphoreType.DMA((NC,)),
        ],
    )(x)


# device time: 482497 ns/iter; 1.8728x vs baseline; 1.0176x over previous
import jax
import jax.numpy as jnp
from jax import lax
from jax.experimental import pallas as pl
from jax.experimental.pallas import tpu as pltpu

M = 32768
N = 1024
H = M // 2
NC = 16
CH = H // NC
LAG = 2


def kernel(x):
    assert x.shape == (M, N), x.shape

    def body(x_hbm, out_hbm, f32_v, bf16_v, cin_sems, cout_sems,
             ssx, rsx, ssy, rsy):
        my_x = lax.axis_index("x")
        my_y = lax.axis_index("y")
        row0 = my_x * M
        peer_row0 = (1 - my_x) * M
        send_half = my_y * H
        other_half = (1 - my_y) * H
        x_nbr = (1 - my_x, my_y)
        y_nbr = (my_x, 1 - my_y)

        def s_row(c):
            return send_half + c * CH

        def o_row(c):
            return other_half + c * CH

        pending_in = {}

        def conv_start(local_r, slot):
            cp = pltpu.make_async_copy(
                x_hbm.at[pl.ds(local_r, CH)], f32_v.at[slot], cin_sems.at[slot]
            )
            cp.start()
            pending_in[slot] = cp

        def conv_finish_store(local_r, slot):
            pending_in.pop(slot).wait()
            bf16_v[slot] = f32_v[slot].astype(jnp.bfloat16)
            cp = pltpu.make_async_copy(
                bf16_v.at[slot],
                out_hbm.at[pl.ds(row0 + local_r, CH)],
                cout_sems.at[slot],
            )
            cp.start()
            cp.wait()

        x_rdma = []
        fwd = []

        def send_x(c):
            r = out_hbm.at[pl.ds(row0 + s_row(c), CH)]
            rdma = pltpu.make_async_remote_copy(
                src_ref=r, dst_ref=r,
                send_sem=ssx.at[c], recv_sem=rsx.at[c],
                device_id=x_nbr, device_id_type=pl.DeviceIdType.MESH,
            )
            rdma.start()
            x_rdma.append(rdma)

        def do_fwd(d):
            x_rdma[d].wait_recv()
            r = out_hbm.at[pl.ds(peer_row0 + send_half + d * CH, CH)]
            f = pltpu.make_async_remote_copy(
                src_ref=r, dst_ref=r,
                send_sem=ssy.at[d], recv_sem=rsy.at[d],
                device_id=y_nbr, device_id_type=pl.DeviceIdType.MESH,
            )
            f.start()
            fwd.append(f)

        def conv_other(d):
            if d + 1 < NC:
                conv_start(o_row(d + 1), 2 + (d + 1) % 2)
            conv_finish_store(o_row(d), 2 + d % 2)

        conv_start(s_row(0), 0)
        conv_start(o_row(0), 2)

        for c in range(NC):
            if c + 1 < NC:
                conv_start(s_row(c + 1), (c + 1) % 2)
            conv_finish_store(s_row(c), c % 2)
            send_x(c)
            if c >= LAG:
                do_fwd(c - LAG)
                conv_other(c - LAG)

        for d in range(NC - LAG, NC):
            do_fwd(d)
            conv_other(d)

        for c in range(NC):
            fwd[c].wait_recv()
        for c in range(NC):
            x_rdma[c].wait_send()
            fwd[c].wait_send()

    return pl.pallas_call(
        body,
        out_shape=jax.ShapeDtypeStruct((2 * M, N), jnp.bfloat16),
        in_specs=[pl.BlockSpec(memory_space=pl.ANY)],
        out_specs=pl.BlockSpec(memory_space=pl.ANY),
        scratch_shapes=[
            pltpu.VMEM((4, CH, N), jnp.float32),
            pltpu.VMEM((4, CH, N), jnp.bfloat16),
            pltpu.SemaphoreType.DMA((4,)),
            pltpu.SemaphoreType.DMA((4,)),
            pltpu.SemaphoreType.DMA((NC,)),
            pltpu.SemaphoreType.DMA((NC,)),
            pltpu.SemaphoreType.DMA((NC,)),
            pltpu.SemaphoreType.DMA((NC,)),
        ],
    )(x)
